# baseline (speedup 1.0000x reference)
import jax
import jax.numpy as jnp
from jax import lax
from jax.experimental import pallas as pl
from jax.experimental.pallas import tpu as pltpu
from jax.experimental.pallas import tpu_sc as plsc


def _nop_body(ids_hbm, out_hbm, buf_v):
    wid = lax.axis_index("s") * 2 + lax.axis_index("c")
    buf_v[pl.ds(0, 16)] = jnp.zeros((16,), jnp.int32)
    pltpu.sync_copy(buf_v, out_hbm.at[wid])


@jax.jit
def _run(ids3):
    mesh = plsc.VectorSubcoreMesh(core_axis_name="c", subcore_axis_name="s",
                                  num_cores=2, num_subcores=16)
    return pl.kernel(
        _nop_body,
        out_type=[jax.ShapeDtypeStruct((32, 16), jnp.int32)],
        mesh=mesh,
        scratch_types=[pltpu.VMEM((16,), jnp.int32)],
        compiler_params=pltpu.CompilerParams(needs_layout_passes=False),
    )(ids3)


def kernel(input_ids, input_part_token_start_idx, shift_batch, label):
    o = _run(input_ids)[0]
    out_ids = jnp.zeros((32, 512), jnp.int32) + o[0, 0]
    return (label, out_ids, out_ids, out_ids, o[:, 0])


# speedup vs baseline: 1.0582x; 1.0582x over previous
import functools

import jax
import jax.numpy as jnp
from jax import lax
from jax.experimental import pallas as pl
from jax.experimental.pallas import tpu as pltpu
from jax.experimental.pallas import tpu_sc as plsc

PAD_ID = 0
CLS_ID = 101
SEP_ID = 102
SEG = 512

_T = 4096
_NROWS = 32
_L = 16
_QBASE = ((_T - 200) // _L) * _L
_NQV = (_T - _QBASE) // _L
_QBUF = (_NQV + 1) * _L
_OUTBUF = 768


def _row_filter_body(ids_hbm, spl_hbm, out_hbm, am_hbm, shift_hbm,
                     row_v, spl_v, out_v, q_v, am_v, shift_v):
    cid = lax.axis_index("c")
    sid = lax.axis_index("s")
    wid = sid * 2 + cid

    pltpu.sync_copy(ids_hbm.at[wid // 4, wid % 4], row_v)
    pltpu.sync_copy(spl_hbm, spl_v.at[pl.ds(0, _NROWS)])

    lanes = jnp.arange(_L, dtype=jnp.int32)
    zeros = jnp.zeros((_L,), dtype=jnp.int32)

    s_val = spl_v[pl.ds(wid, _L)][0]

    def zf_out(k, _):
        out_v[pl.ds(k * _L, _L)] = zeros
        return 0

    def zf_q(k, _):
        q_v[pl.ds(k * _L, _L)] = zeros
        return 0

    lax.fori_loop(0, _OUTBUF // _L, zf_out, 0)
    lax.fori_loop(0, _QBUF // _L, zf_q, 0)

    def qstep(k, lq):
        v = row_v[pl.ds(_QBASE + k * _L, _L)]
        pos = _QBASE + k * _L + lanes
        m = (pos >= s_val) & (v != PAD_ID) & (v != CLS_ID)
        plsc.store_compressed(q_v.at[pl.ds(lq, _L)], v, mask=m)
        return lq + plsc.all_reduce_population_count(m)[0]

    len_q = lax.fori_loop(0, _NQV, qstep, jnp.int32(0))

    seg_target = SEG - 1 - len_q

    nmax = (s_val + _L - 1) // _L

    def cond(carry):
        i, cnt = carry
        return (i < nmax) & (cnt < seg_target)

    def body(carry):
        i, cnt = carry
        v = row_v[pl.ds(i * _L, _L)]
        pos = i * _L + lanes
        m = ((pos < s_val) & (v != PAD_ID) & (v != CLS_ID) & (v != SEP_ID))
        plsc.store_compressed(out_v.at[pl.ds(1 + cnt, _L)], v, mask=m)
        return i + 1, cnt + plsc.all_reduce_population_count(m)[0]

    _, cnt = lax.while_loop(cond, body, (jnp.int32(0), jnp.int32(0)))
    seg_len = jnp.minimum(cnt, seg_target)

    def splice(k, _):
        out_v[pl.ds(1 + seg_len + k * _L, _L)] = q_v[pl.ds(k * _L, _L)]
        return 0

    lax.fori_loop(0, _QBUF // _L, splice, 0)

    v0 = out_v[pl.ds(0, _L)]
    out_v[pl.ds(0, _L)] = jnp.where(lanes == 0, jnp.int32(CLS_ID), v0)

    def amstep(k, _):
        v = out_v[pl.ds(k * _L, _L)]
        am_v[pl.ds(k * _L, _L)] = (v != PAD_ID).astype(jnp.int32)
        return 0

    lax.fori_loop(0, SEG // _L, amstep, 0)

    shift_v[pl.ds(0, _L)] = jnp.full((_L,), 1, jnp.int32) * seg_len

    pltpu.sync_copy(out_v.at[pl.ds(0, SEG)], out_hbm.at[wid])
    pltpu.sync_copy(am_v, am_hbm.at[wid])
    pltpu.sync_copy(shift_v, shift_hbm.at[wid])


@jax.jit
def _run(ids3, spl):
    mesh = plsc.VectorSubcoreMesh(core_axis_name="c", subcore_axis_name="s",
                                  num_cores=2, num_subcores=16)
    out_ids, am, shift = pl.kernel(
        _row_filter_body,
        out_type=[
            jax.ShapeDtypeStruct((_NROWS, SEG), jnp.int32),
            jax.ShapeDtypeStruct((_NROWS, SEG), jnp.int32),
            jax.ShapeDtypeStruct((_NROWS, _L), jnp.int32),
        ],
        mesh=mesh,
        scratch_types=[
            pltpu.VMEM((_T,), jnp.int32),
            pltpu.VMEM((_NROWS + _L,), jnp.int32),
            pltpu.VMEM((_OUTBUF,), jnp.int32),
            pltpu.VMEM((_QBUF,), jnp.int32),
            pltpu.VMEM((SEG,), jnp.int32),
            pltpu.VMEM((_L,), jnp.int32),
        ],
        compiler_params=pltpu.CompilerParams(needs_layout_passes=False),
    )(ids3, spl)
    return out_ids, am, shift


def kernel(input_ids, input_part_token_start_idx, shift_batch, label):
    B, C, T = input_ids.shape
    spl = input_part_token_start_idx.reshape(B * C).astype(jnp.int32)
    out_ids, attention_mask, shift16 = _run(input_ids, spl)
    new_shift = shift16[:, 0]
    token_type_ids = jnp.zeros_like(out_ids)
    return (label, out_ids, attention_mask, token_type_ids, new_shift)
